# baseline (speedup 1.0000x reference)
import functools

import jax
import jax.numpy as jnp
from jax import lax
from jax.experimental import pallas as pl
from jax.experimental.pallas import tpu as pltpu
from jax.experimental.pallas import tpu_sc as plsc

NC = 2
NS = 16
NW = NC * NS
LANES = 16

BATCH = 16384
DIM = 64
CHUNKS = DIM // LANES
ROWS_PER_W = BATCH // NW
IDX_CHUNK = 128
N_STREAMS = ROWS_PER_W // IDX_CHUNK
ROW_UNROLL = 8


def _rsqrt(ss):
    i = lax.bitcast_convert_type(ss, jnp.int32)
    i = jnp.int32(0x5F3759DF) - (i >> 1)
    y = lax.bitcast_convert_type(i, jnp.float32)
    ssh = 0.5 * ss
    for _ in range(2):
        y = y * (1.5 - ssh * y * y)
    y = y * (1.5 - ssh * y * y)
    return y


_GATHER_DNUMS = lax.GatherDimensionNumbers(
    offset_dims=(), collapsed_slice_dims=(0,), start_index_map=(0,))


def _xlane(v, idx):
    return lax.gather(v, idx[:, None], _GATHER_DNUMS, slice_sizes=(1,),
                      mode=lax.GatherScatterMode.PROMISE_IN_BOUNDS)


def _normalize_rows(rows):
    lanes = lax.iota(jnp.int32, LANES)

    def body(g, carry):
        for r in range(ROW_UNROLL):
            row = g * ROW_UNROLL + r
            x = [rows[row, pl.ds(c * LANES, LANES)] for c in range(CHUNKS)]
            p = x[0] * x[0]
            for c in range(1, CHUNKS):
                p = p + x[c] * x[c]
            for sh in (8, 4, 2, 1):
                p = p + _xlane(p, lanes ^ sh)
            y = _rsqrt(jnp.maximum(p, 1e-30))
            for c in range(CHUNKS):
                rows[row, pl.ds(c * LANES, LANES)] = x[c] * y
        return carry
    lax.fori_loop(0, ROWS_PER_W // ROW_UNROLL, body, 0)


@functools.partial(
    pl.kernel,
    mesh=plsc.VectorSubcoreMesh(core_axis_name="c", subcore_axis_name="s"),
    compiler_params=pltpu.CompilerParams(use_tc_tiling_on_sc=False),
    out_type=[
        jax.ShapeDtypeStruct((BATCH, DIM), jnp.float32),
        jax.ShapeDtypeStruct((BATCH, DIM), jnp.float32),
    ],
    scratch_types=[
        pltpu.VMEM((ROWS_PER_W,), jnp.int32),
        pltpu.VMEM((ROWS_PER_W,), jnp.int32),
        pltpu.VMEM((ROWS_PER_W, DIM), jnp.float32),
        pltpu.VMEM((ROWS_PER_W, DIM), jnp.float32),
        pltpu.SemaphoreType.DMA,
        pltpu.SemaphoreType.DMA,
        pltpu.SemaphoreType.DMA,
    ],
)
def _embed_norm(user_ids, item_ids, user_table, item_table,
                out_u, out_i, idx_u, idx_i, rows_u, rows_i,
                sem_u, sem_i, sem_out):
    wid = lax.axis_index("s") * NC + lax.axis_index("c")
    base = wid * ROWS_PER_W

    pltpu.sync_copy(user_ids.at[pl.ds(base, ROWS_PER_W)], idx_u)
    pltpu.sync_copy(item_ids.at[pl.ds(base, ROWS_PER_W)], idx_i)

    waits_u = []
    waits_i = []
    for j in range(N_STREAMS):
        sl = pl.ds(j * IDX_CHUNK, IDX_CHUNK)
        waits_u.append(pltpu.async_copy(
            user_table.at[idx_u.at[sl]], rows_u.at[sl], sem_u))
    for j in range(N_STREAMS):
        sl = pl.ds(j * IDX_CHUNK, IDX_CHUNK)
        waits_i.append(pltpu.async_copy(
            item_table.at[idx_i.at[sl]], rows_i.at[sl], sem_i))

    for w in waits_u:
        w.wait()
    _normalize_rows(rows_u)
    out_wait = pltpu.async_copy(rows_u, out_u.at[pl.ds(base, ROWS_PER_W)],
                                sem_out)

    for w in waits_i:
        w.wait()
    _normalize_rows(rows_i)
    pltpu.sync_copy(rows_i, out_i.at[pl.ds(base, ROWS_PER_W)])
    out_wait.wait()


def kernel(user_ids, item_ids, user_table, item_table):
    out_u, out_i = _embed_norm(user_ids.astype(jnp.int32),
                               item_ids.astype(jnp.int32),
                               user_table, item_table)
    return out_u, out_i


# speedup vs baseline: 1.0022x; 1.0022x over previous
import functools

import jax
import jax.numpy as jnp
from jax import lax
from jax.experimental import pallas as pl
from jax.experimental.pallas import tpu as pltpu
from jax.experimental.pallas import tpu_sc as plsc

NC = 2
NS = 16
NW = NC * NS
LANES = 16

BATCH = 16384
DIM = 64
CHUNKS = DIM // LANES
ROWS_PER_W = BATCH // NW
IDX_CHUNK = 128
N_STREAMS = ROWS_PER_W // IDX_CHUNK
ROW_UNROLL = 8


def _rsqrt(ss):
    i = lax.bitcast_convert_type(ss, jnp.int32)
    i = jnp.int32(0x5F3759DF) - (i >> 1)
    y = lax.bitcast_convert_type(i, jnp.float32)
    ssh = 0.5 * ss
    for _ in range(2):
        y = y * (1.5 - ssh * y * y)
    y = y * (1.5 - ssh * y * y)
    return y


_GATHER_DNUMS = lax.GatherDimensionNumbers(
    offset_dims=(), collapsed_slice_dims=(0,), start_index_map=(0,))


def _xlane(v, idx):
    return lax.gather(v, idx[:, None], _GATHER_DNUMS, slice_sizes=(1,),
                      mode=lax.GatherScatterMode.PROMISE_IN_BOUNDS)


def _normalize_chunk(rows, chunk):
    lanes = lax.iota(jnp.int32, LANES)

    def body(g, carry):
        for r in range(ROW_UNROLL):
            row = chunk * IDX_CHUNK + g * ROW_UNROLL + r
            x = [rows[row, pl.ds(c * LANES, LANES)] for c in range(CHUNKS)]
            p = x[0] * x[0]
            for c in range(1, CHUNKS):
                p = p + x[c] * x[c]
            for sh in (8, 4, 2, 1):
                p = p + _xlane(p, lanes ^ sh)
            y = _rsqrt(jnp.maximum(p, 1e-30))
            for c in range(CHUNKS):
                rows[row, pl.ds(c * LANES, LANES)] = x[c] * y
        return carry
    lax.fori_loop(0, IDX_CHUNK // ROW_UNROLL, body, 0)


@functools.partial(
    pl.kernel,
    mesh=plsc.VectorSubcoreMesh(core_axis_name="c", subcore_axis_name="s"),
    compiler_params=pltpu.CompilerParams(use_tc_tiling_on_sc=False),
    out_type=[
        jax.ShapeDtypeStruct((BATCH, DIM), jnp.float32),
        jax.ShapeDtypeStruct((BATCH, DIM), jnp.float32),
    ],
    scratch_types=[
        pltpu.VMEM((ROWS_PER_W,), jnp.int32),
        pltpu.VMEM((ROWS_PER_W,), jnp.int32),
        pltpu.VMEM((ROWS_PER_W, DIM), jnp.float32),
        pltpu.VMEM((ROWS_PER_W, DIM), jnp.float32),
    ] + [pltpu.SemaphoreType.DMA] * (2 * N_STREAMS + 1),
)
def _embed_norm(user_ids, item_ids, user_table, item_table,
                out_u, out_i, idx_u, idx_i, rows_u, rows_i,
                *sems):
    gather_sems = sems[:2 * N_STREAMS]
    sem_out = sems[2 * N_STREAMS]
    wid = lax.axis_index("s") * NC + lax.axis_index("c")
    base = wid * ROWS_PER_W

    pltpu.sync_copy(user_ids.at[pl.ds(base, ROWS_PER_W)], idx_u)
    gathers = []
    for j in range(N_STREAMS):
        sl = pl.ds(j * IDX_CHUNK, IDX_CHUNK)
        gathers.append(pltpu.async_copy(
            user_table.at[idx_u.at[sl]], rows_u.at[sl], gather_sems[j]))
    pltpu.sync_copy(item_ids.at[pl.ds(base, ROWS_PER_W)], idx_i)
    for j in range(N_STREAMS):
        sl = pl.ds(j * IDX_CHUNK, IDX_CHUNK)
        gathers.append(pltpu.async_copy(
            item_table.at[idx_i.at[sl]], rows_i.at[sl],
            gather_sems[N_STREAMS + j]))

    out_waits = []
    for j in range(N_STREAMS):
        gathers[j].wait()
        _normalize_chunk(rows_u, j)
        sl = pl.ds(j * IDX_CHUNK, IDX_CHUNK)
        out_waits.append(pltpu.async_copy(
            rows_u.at[sl], out_u.at[pl.ds(base + j * IDX_CHUNK, IDX_CHUNK)],
            sem_out))
    for j in range(N_STREAMS):
        gathers[N_STREAMS + j].wait()
        _normalize_chunk(rows_i, j)
        sl = pl.ds(j * IDX_CHUNK, IDX_CHUNK)
        out_waits.append(pltpu.async_copy(
            rows_i.at[sl], out_i.at[pl.ds(base + j * IDX_CHUNK, IDX_CHUNK)],
            sem_out))
    for w in out_waits:
        w.wait()


def kernel(user_ids, item_ids, user_table, item_table):
    out_u, out_i = _embed_norm(user_ids.astype(jnp.int32),
                               item_ids.astype(jnp.int32),
                               user_table, item_table)
    return out_u, out_i


# speedup vs baseline: 2.3785x; 2.3732x over previous
import functools

import jax
import jax.numpy as jnp
from jax import lax
from jax.experimental import pallas as pl
from jax.experimental.pallas import tpu as pltpu
from jax.experimental.pallas import tpu_sc as plsc

NC = 2
NS = 16
NW = NC * NS
LANES = 16

NB_ROWS = 1000000
BATCH = 16384
DIM = 64
CHUNKS = DIM // LANES
ROWS_PER_W = BATCH // NW
NBUF = 4
BLK = 128

_GATHER_DNUMS = lax.GatherDimensionNumbers(
    offset_dims=(), collapsed_slice_dims=(0,), start_index_map=(0,))


def _xlane(v, idx):
    return lax.gather(v, idx[:, None], _GATHER_DNUMS, slice_sizes=(1,),
                      mode=lax.GatherScatterMode.PROMISE_IN_BOUNDS)


def _rsqrt(ss):
    i = lax.bitcast_convert_type(ss, jnp.int32)
    i = jnp.int32(0x5F3759DF) - (i >> 1)
    y = lax.bitcast_convert_type(i, jnp.float32)
    ssh = 0.5 * ss
    for _ in range(3):
        y = y * (1.5 - ssh * y * y)
    return y


def _mesh():
    return plsc.VectorSubcoreMesh(core_axis_name="c", subcore_axis_name="s")


@functools.partial(
    pl.kernel,
    mesh=_mesh(),
    out_type=[
        jax.ShapeDtypeStruct((BATCH, DIM), jnp.float32),
        jax.ShapeDtypeStruct((BATCH, DIM), jnp.float32),
    ],
    compiler_params=pltpu.CompilerParams(use_tc_tiling_on_sc=True,
                                         needs_layout_passes=False),
    scratch_types=[
        pltpu.VMEM((ROWS_PER_W + LANES,), jnp.int32),
        pltpu.VMEM((NBUF, DIM, BLK), jnp.float32),
        pltpu.VMEM((ROWS_PER_W, DIM), jnp.float32),
    ] + [pltpu.SemaphoreType.DMA] * NBUF + [pltpu.SemaphoreType.DMA],
)
def _embed_norm(user_ids, item_ids, user_table_t, item_table_t,
                out_u, out_i, idx_v, ring, staging, *sems):
    ring_sems = sems[:NBUF]
    sem_out = sems[NBUF]
    wid = lax.axis_index("s") * NC + lax.axis_index("c")
    base = wid * ROWS_PER_W
    lanes = lax.iota(jnp.int32, LANES)
    fidx = [lanes + c * LANES for c in range(CHUNKS)]

    def run_table(tab_t, ids, out):
        pltpu.sync_copy(ids.at[pl.ds(base, ROWS_PER_W)],
                        idx_v.at[pl.ds(0, ROWS_PER_W)])

        def read_idx(j):
            return idx_v[pl.ds(j, LANES)][0]

        def blk_start(iv):
            return pl.multiple_of(iv & jnp.int32(~(BLK - 1)), BLK)

        def fire(j, slot):
            start = blk_start(read_idx(j))
            return pltpu.async_copy(
                tab_t.at[:, pl.ds(start, BLK)], ring.at[slot],
                ring_sems[slot])

        def process(j, slot):
            iv = read_idx(j)
            colv = jnp.full((LANES,), iv & jnp.int32(BLK - 1), jnp.int32)
            x = [plsc.load_gather(ring.at[slot], [fidx[c], colv])
                 for c in range(CHUNKS)]
            p = x[0] * x[0]
            for c in range(1, CHUNKS):
                p = p + x[c] * x[c]
            for sh in (8, 4, 2, 1):
                p = p + _xlane(p, lanes ^ sh)
            y = _rsqrt(jnp.maximum(p, 1e-30))
            for c in range(CHUNKS):
                staging[j, pl.ds(c * LANES, LANES)] = x[c] * y

        waits = [fire(jnp.int32(s), s) for s in range(NBUF)]

        def group(g, carry):
            for s in range(NBUF):
                j = g * NBUF + s
                waits[s].wait()
                process(j, s)
                fire(j + NBUF, s)
            return carry
        lax.fori_loop(0, ROWS_PER_W // NBUF - 1, group, 0)
        for s in range(NBUF):
            j = (ROWS_PER_W // NBUF - 1) * NBUF + s
            waits[s].wait()
            process(jnp.int32(j), s)

        pltpu.async_copy(staging, out.at[pl.ds(base, ROWS_PER_W)],
                         sem_out).wait()

    run_table(user_table_t, user_ids, out_u)
    run_table(item_table_t, item_ids, out_i)


def kernel(user_ids, item_ids, user_table, item_table):
    out_u, out_i = _embed_norm(user_ids.astype(jnp.int32),
                               item_ids.astype(jnp.int32),
                               user_table.T, item_table.T)
    return out_u, out_i


# speedup vs baseline: 2.7737x; 1.1661x over previous
import functools

import jax
import jax.numpy as jnp
from jax import lax
from jax.experimental import pallas as pl
from jax.experimental.pallas import tpu as pltpu
from jax.experimental.pallas import tpu_sc as plsc

NC = 2
NS = 16
NW = NC * NS
LANES = 16

NB_ROWS = 1000000
BATCH = 16384
DIM = 64
CHUNKS = DIM // LANES
ROWS_PER_W = BATCH // NW
NBUF = 8
BLK = 128
STG = 128
NCHUNK = ROWS_PER_W // STG

_GATHER_DNUMS = lax.GatherDimensionNumbers(
    offset_dims=(), collapsed_slice_dims=(0,), start_index_map=(0,))


def _xlane(v, idx):
    return lax.gather(v, idx[:, None], _GATHER_DNUMS, slice_sizes=(1,),
                      mode=lax.GatherScatterMode.PROMISE_IN_BOUNDS)


def _rsqrt(ss):
    i = lax.bitcast_convert_type(ss, jnp.int32)
    i = jnp.int32(0x5F3759DF) - (i >> 1)
    y = lax.bitcast_convert_type(i, jnp.float32)
    ssh = 0.5 * ss
    for _ in range(3):
        y = y * (1.5 - ssh * y * y)
    return y


def _mesh():
    return plsc.VectorSubcoreMesh(core_axis_name="c", subcore_axis_name="s")


@functools.partial(
    pl.kernel,
    mesh=_mesh(),
    out_type=[
        jax.ShapeDtypeStruct((BATCH, DIM), jnp.float32),
        jax.ShapeDtypeStruct((BATCH, DIM), jnp.float32),
    ],
    compiler_params=pltpu.CompilerParams(use_tc_tiling_on_sc=True,
                                         needs_layout_passes=False),
    scratch_types=[
        pltpu.VMEM((ROWS_PER_W + LANES,), jnp.int32),
        pltpu.VMEM((NBUF, DIM, BLK), jnp.float32),
        pltpu.VMEM((2, STG, DIM), jnp.float32),
    ] + [pltpu.SemaphoreType.DMA] * (NBUF + 2),
)
def _embed_norm(user_ids, item_ids, user_table_t, item_table_t,
                out_u, out_i, idx_v, ring, staging, *sems):
    ring_sems = sems[:NBUF]
    out_sems = sems[NBUF:]
    wid = lax.axis_index("s") * NC + lax.axis_index("c")
    base = wid * ROWS_PER_W
    lanes = lax.iota(jnp.int32, LANES)
    fidx = [lanes + c * LANES for c in range(CHUNKS)]

    def run_table(tab_t, ids, out):
        pltpu.sync_copy(ids.at[pl.ds(base, ROWS_PER_W)],
                        idx_v.at[pl.ds(0, ROWS_PER_W)])

        def read_idx(j):
            return idx_v[pl.ds(j, LANES)][0]

        def blk_start(iv):
            return pl.multiple_of(iv & jnp.int32(~(BLK - 1)), BLK)

        def fire(j, slot):
            start = blk_start(read_idx(j))
            return pltpu.async_copy(
                tab_t.at[:, pl.ds(start, BLK)], ring.at[slot],
                ring_sems[slot])

        def process(j, slot, bank):
            iv = read_idx(j)
            colv = jnp.full((LANES,), iv & jnp.int32(BLK - 1), jnp.int32)
            x = [plsc.load_gather(ring.at[slot], [fidx[c], colv])
                 for c in range(CHUNKS)]
            p = x[0] * x[0]
            for c in range(1, CHUNKS):
                p = p + x[c] * x[c]
            for sh in (8, 4, 2, 1):
                p = p + _xlane(p, lanes ^ sh)
            y = _rsqrt(jnp.maximum(p, 1e-30))
            jl = j % STG if isinstance(j, int) else j & jnp.int32(STG - 1)
            for c in range(CHUNKS):
                staging[bank, jl, pl.ds(c * LANES, LANES)] = x[c] * y

        waits = [fire(jnp.int32(s), s) for s in range(NBUF)]
        out_waits = [None, None]
        for k in range(NCHUNK):
            bank = k % 2
            if out_waits[bank] is not None:
                out_waits[bank].wait()

            def group(g, carry, k=k, bank=bank):
                for s in range(NBUF):
                    j = jnp.int32(k * STG) + g * NBUF + s
                    waits[s].wait()
                    process(j, s, bank)
                    fire(j + NBUF, s)
                return carry
            last = (k == NCHUNK - 1)
            ngrp = STG // NBUF - (1 if last else 0)
            lax.fori_loop(0, ngrp, group, 0)
            if last:
                for s in range(NBUF):
                    j = k * STG + (STG // NBUF - 1) * NBUF + s
                    waits[s].wait()
                    process(j, s, bank)
            out_waits[bank] = pltpu.async_copy(
                staging.at[bank], out.at[pl.ds(base + k * STG, STG)],
                out_sems[bank])
        for w in out_waits:
            w.wait()

    run_table(user_table_t, user_ids, out_u)
    run_table(item_table_t, item_ids, out_i)


def kernel(user_ids, item_ids, user_table, item_table):
    out_u, out_i = _embed_norm(user_ids.astype(jnp.int32),
                               item_ids.astype(jnp.int32),
                               user_table.T, item_table.T)
    return out_u, out_i
